# baseline (speedup 1.0000x reference)
import functools

import jax
import jax.numpy as jnp
from jax import lax
from jax.experimental import pallas as pl
from jax.experimental.pallas import tpu as pltpu
from jax.experimental.pallas import tpu_sc as plsc

B = 16
T = 128
V = 32000
NSUB = 16
TPW = T // NSUB

_LN2 = 0.6931471805599453
_SQRT2 = 1.4142135623730951


def _neg_log(x):
    bits = lax.bitcast_convert_type(x, jnp.int32)
    e = lax.shift_right_logical(bits, 23) - 127
    mbits = jnp.bitwise_or(jnp.bitwise_and(bits, 0x007FFFFF), 0x3F800000)
    m = lax.bitcast_convert_type(mbits, jnp.float32)
    big = m > _SQRT2
    m = jnp.where(big, m * 0.5, m)
    ef = e.astype(jnp.float32) + jnp.where(big, 1.0, 0.0)
    z = (m - 1.0) / (m + 1.0)
    z2 = z * z
    p = 1.0 + z2 * (1.0 / 3.0 + z2 * (1.0 / 5.0 + z2 * (1.0 / 7.0 + z2 * (1.0 / 9.0))))
    return -(2.0 * z * p + ef * _LN2)


def _rl_loss_body(chosen_t, mask_t, delta, probs_flat, out,
                  cv, mv, idxv, gv, lossv, ntokv, bigl, bign, dv, ov,
                  sh_loss, sh_ntok, sem):
    w = lax.axis_index("s")
    base_t = w * TPW

    pltpu.sync_copy(chosen_t.at[pl.ds(base_t, TPW)], cv)
    pltpu.sync_copy(mask_t.at[pl.ds(base_t, TPW)], mv)

    lane_b = lax.iota(jnp.int32, 16)
    for j in range(TPW):
        flat = (lane_b * T + (base_t + j)) * V + cv[j]
        idxv[pl.ds(16 * j, 16)] = flat

    pltpu.async_copy(probs_flat.at[idxv], gv, sem).wait()

    lacc = jnp.zeros((16,), jnp.float32)
    nacc = jnp.zeros((16,), jnp.float32)
    for j in range(TPW):
        m = mv[j]
        lacc = lacc + _neg_log(gv[pl.ds(16 * j, 16)]) * m
        nacc = nacc + m

    lossv[...] = lacc
    ntokv[...] = nacc
    pltpu.sync_copy(lossv, sh_loss.at[pl.ds(16 * w, 16)])
    pltpu.sync_copy(ntokv, sh_ntok.at[pl.ds(16 * w, 16)])
    plsc.subcore_barrier()

    @pl.when(w == 0)
    def _finalize():
        pltpu.sync_copy(sh_loss, bigl)
        pltpu.sync_copy(sh_ntok, bign)
        pltpu.sync_copy(delta, dv)
        lsum = jnp.zeros((16,), jnp.float32)
        nsum = jnp.zeros((16,), jnp.float32)
        for k in range(NSUB):
            lsum = lsum + bigl[pl.ds(16 * k, 16)]
            nsum = nsum + bign[pl.ds(16 * k, 16)]
        ov[...] = lsum * dv[...] / nsum
        pltpu.sync_copy(ov, out)


@functools.cache
def _build_rl_loss_sc():
    return pl.kernel(
        _rl_loss_body,
        out_type=jax.ShapeDtypeStruct((B,), jnp.float32),
        mesh=plsc.VectorSubcoreMesh(core_axis_name="c", subcore_axis_name="s",
                                    num_cores=1),
        scratch_types=[
            pltpu.VMEM((TPW, 16), jnp.int32),
            pltpu.VMEM((TPW, 16), jnp.float32),
            pltpu.VMEM((TPW * 16,), jnp.int32),
            pltpu.VMEM((TPW * 16,), jnp.float32),
            pltpu.VMEM((16,), jnp.float32),
            pltpu.VMEM((16,), jnp.float32),
            pltpu.VMEM((NSUB * 16,), jnp.float32),
            pltpu.VMEM((NSUB * 16,), jnp.float32),
            pltpu.VMEM((16,), jnp.float32),
            pltpu.VMEM((16,), jnp.float32),
            pltpu.VMEM_SHARED((NSUB * 16,), jnp.float32),
            pltpu.VMEM_SHARED((NSUB * 16,), jnp.float32),
            pltpu.SemaphoreType.DMA,
        ],
    )


def kernel(chosen_tokens, probs, time_step_mask, delta_reward):
    chosen_t = jnp.transpose(chosen_tokens.astype(jnp.int32))
    mask_t = jnp.transpose(time_step_mask)
    probs_flat = jnp.reshape(probs, (-1,))
    return _build_rl_loss_sc()(chosen_t, mask_t, delta_reward, probs_flat)


# speedup vs baseline: 8.0956x; 8.0956x over previous
import functools

import jax
import jax.numpy as jnp
from jax import lax
from jax.experimental import pallas as pl
from jax.experimental.pallas import tpu as pltpu
from jax.experimental.pallas import tpu_sc as plsc

B = 16
T = 128
V = 32000
NG = T // 16

_LN2 = 0.6931471805599453
_SQRT2 = 1.4142135623730951


def _neg_log(x):
    bits = lax.bitcast_convert_type(x, jnp.int32)
    e = lax.shift_right_logical(bits, 23) - 127
    mbits = jnp.bitwise_or(jnp.bitwise_and(bits, 0x007FFFFF), 0x3F800000)
    m = lax.bitcast_convert_type(mbits, jnp.float32)
    big = m > _SQRT2
    m = jnp.where(big, m * 0.5, m)
    ef = e.astype(jnp.float32) + jnp.where(big, 1.0, 0.0)
    z = (m - 1.0) / (m + 1.0)
    z2 = z * z
    p = 1.0 + z2 * (1.0 / 3.0 + z2 * (1.0 / 5.0 + z2 * (1.0 / 7.0 + z2 * (1.0 / 9.0))))
    return -(2.0 * z * p + ef * _LN2)


def _rl_loss_body(chosen, mask, delta, probs, out,
                  cvb, mvb, buf, stgl, stgn, bigl, bign, dv, ov,
                  sh_loss, sh_ntok, sem):
    w = lax.axis_index("s")

    pltpu.sync_copy(chosen.at[w], cvb)
    pltpu.sync_copy(mask.at[w], mvb)

    handles = []
    for g in range(NG):
        cv = cvb[pl.ds(16 * g, 16)]
        for j in range(16):
            c = cv[j]
            v0 = pl.multiple_of(jnp.bitwise_and(c, 0x7FF0), 16)
            t = 16 * g + j
            handles.append(
                pltpu.async_copy(probs.at[w, t, pl.ds(v0, 16)], buf.at[t], sem))
    for h in handles:
        h.wait()

    lane = lax.iota(jnp.int32, 16)
    acc = jnp.zeros((16,), jnp.float32)
    nacc = jnp.zeros((16,), jnp.float32)
    for g in range(NG):
        cv = cvb[pl.ds(16 * g, 16)]
        cols = jnp.bitwise_and(cv, 15)
        m = mvb[pl.ds(16 * g, 16)]
        sel = jnp.zeros((16,), jnp.float32)
        for j in range(16):
            chunk = buf[16 * g + j]
            gj = chunk[jnp.full((16,), cols[j], jnp.int32)]
            sel = jnp.where(lane == j, gj, sel)
        acc = acc + _neg_log(sel) * m
        nacc = nacc + m

    def _lane_sum(x):
        for sh in (8, 4, 2, 1):
            x = x + x[jnp.bitwise_xor(lane, sh)]
        return x

    stgl[...] = jnp.where(lane == w, _lane_sum(acc), 0.0)
    stgn[...] = jnp.where(lane == w, _lane_sum(nacc), 0.0)
    pltpu.sync_copy(stgl, sh_loss.at[pl.ds(16 * w, 16)])
    pltpu.sync_copy(stgn, sh_ntok.at[pl.ds(16 * w, 16)])
    plsc.subcore_barrier()

    @pl.when(w == 0)
    def _finalize():
        pltpu.sync_copy(sh_loss, bigl)
        pltpu.sync_copy(sh_ntok, bign)
        pltpu.sync_copy(delta, dv)
        lt = jnp.zeros((16,), jnp.float32)
        nt = jnp.zeros((16,), jnp.float32)
        for k in range(B):
            lt = lt + bigl[pl.ds(16 * k, 16)]
            nt = nt + bign[pl.ds(16 * k, 16)]
        ov[...] = lt * dv[...] / nt
        pltpu.sync_copy(ov, out)


@functools.cache
def _build_rl_loss_sc():
    return pl.kernel(
        _rl_loss_body,
        out_type=jax.ShapeDtypeStruct((B,), jnp.float32),
        mesh=plsc.VectorSubcoreMesh(core_axis_name="c", subcore_axis_name="s",
                                    num_cores=1),
        scratch_types=[
            pltpu.VMEM((T,), jnp.int32),
            pltpu.VMEM((T,), jnp.float32),
            pltpu.VMEM((T, 16), jnp.float32),
            pltpu.VMEM((16,), jnp.float32),
            pltpu.VMEM((16,), jnp.float32),
            pltpu.VMEM((B * 16,), jnp.float32),
            pltpu.VMEM((B * 16,), jnp.float32),
            pltpu.VMEM((16,), jnp.float32),
            pltpu.VMEM((16,), jnp.float32),
            pltpu.VMEM_SHARED((B * 16,), jnp.float32),
            pltpu.VMEM_SHARED((B * 16,), jnp.float32),
            pltpu.SemaphoreType.DMA,
        ],
    )


def kernel(chosen_tokens, probs, time_step_mask, delta_reward):
    return _build_rl_loss_sc()(chosen_tokens.astype(jnp.int32), time_step_mask,
                               delta_reward, probs)


# speedup vs baseline: 8.4812x; 1.0476x over previous
import functools

import jax
import jax.numpy as jnp
from jax import lax
from jax.experimental import pallas as pl
from jax.experimental.pallas import tpu as pltpu
from jax.experimental.pallas import tpu_sc as plsc

B = 16
T = 128
V = 32000
TPW = T // 2
NG = TPW // 16

_LN2 = 0.6931471805599453
_SQRT2 = 1.4142135623730951


def _neg_log(x):
    bits = lax.bitcast_convert_type(x, jnp.int32)
    e = lax.shift_right_logical(bits, 23) - 127
    mbits = jnp.bitwise_or(jnp.bitwise_and(bits, 0x007FFFFF), 0x3F800000)
    m = lax.bitcast_convert_type(mbits, jnp.float32)
    big = m > _SQRT2
    m = jnp.where(big, m * 0.5, m)
    ef = e.astype(jnp.float32) + jnp.where(big, 1.0, 0.0)
    z = (m - 1.0) / (m + 1.0)
    z2 = z * z
    p = 1.0 + z2 * (1.0 / 3.0 + z2 * (1.0 / 5.0 + z2 * (1.0 / 7.0 + z2 * (1.0 / 9.0))))
    return -(2.0 * z * p + ef * _LN2)


def _rl_loss_body(chosen, mask, delta, probs, out,
                  cvb, mvb, buf, stgl, stgn, bigl, bign, dv, ov,
                  sh_loss, sh_ntok, sem):
    c = lax.axis_index("c")
    s = lax.axis_index("s")
    b = 8 * c + lax.shift_right_logical(s, 1)
    base_t = TPW * jnp.bitwise_and(s, 1)

    pltpu.sync_copy(chosen.at[b, pl.ds(base_t, TPW)], cvb)
    mask_h = pltpu.async_copy(mask.at[b, pl.ds(base_t, TPW)], mvb, sem)

    handles = []
    for g in range(NG):
        cv = cvb[pl.ds(16 * g, 16)]
        for j in range(16):
            cj = cv[j]
            v0 = pl.multiple_of(jnp.bitwise_and(cj, 0x7FF0), 16)
            t = 16 * g + j
            handles.append(
                pltpu.async_copy(probs.at[b, base_t + t, pl.ds(v0, 16)],
                                 buf.at[t], sem))
    mask_h.wait()
    for h in handles:
        h.wait()

    lane = lax.iota(jnp.int32, 16)
    acc = jnp.zeros((16,), jnp.float32)
    nacc = jnp.zeros((16,), jnp.float32)
    for g in range(NG):
        cv = cvb[pl.ds(16 * g, 16)]
        cols = jnp.bitwise_and(cv, 15)
        m = mvb[pl.ds(16 * g, 16)]
        sel = jnp.zeros((16,), jnp.float32)
        for j in range(16):
            chunk = buf[16 * g + j]
            gj = chunk[jnp.full((16,), cols[j], jnp.int32)]
            sel = jnp.where(lane == j, gj, sel)
        acc = acc + _neg_log(sel) * m
        nacc = nacc + m

    def _lane_sum(x):
        for sh in (8, 4, 2, 1):
            x = x + x[jnp.bitwise_xor(lane, sh)]
        return x

    stgl[...] = jnp.where(lane == b, _lane_sum(acc), 0.0)
    stgn[...] = jnp.where(lane == b, _lane_sum(nacc), 0.0)
    pltpu.sync_copy(stgl, sh_loss.at[pl.ds(16 * s, 16)])
    pltpu.sync_copy(stgn, sh_ntok.at[pl.ds(16 * s, 16)])
    plsc.subcore_barrier()

    @pl.when(s == 0)
    def _finalize():
        pltpu.sync_copy(sh_loss, bigl)
        pltpu.sync_copy(sh_ntok, bign)
        pltpu.sync_copy(delta, dv)
        lt = jnp.zeros((16,), jnp.float32)
        nt = jnp.zeros((16,), jnp.float32)
        for k in range(16):
            lt = lt + bigl[pl.ds(16 * k, 16)]
            nt = nt + bign[pl.ds(16 * k, 16)]
        ov[...] = lt * dv[...] / nt
        half = pl.multiple_of(8 * c, 8)
        pltpu.sync_copy(ov.at[pl.ds(half, 8)], out.at[pl.ds(half, 8)])


@functools.cache
def _build_rl_loss_sc():
    return pl.kernel(
        _rl_loss_body,
        out_type=jax.ShapeDtypeStruct((B,), jnp.float32),
        mesh=plsc.VectorSubcoreMesh(core_axis_name="c", subcore_axis_name="s",
                                    num_cores=2),
        scratch_types=[
            pltpu.VMEM((TPW,), jnp.int32),
            pltpu.VMEM((TPW,), jnp.float32),
            pltpu.VMEM((TPW, 16), jnp.float32),
            pltpu.VMEM((16,), jnp.float32),
            pltpu.VMEM((16,), jnp.float32),
            pltpu.VMEM((16 * 16,), jnp.float32),
            pltpu.VMEM((16 * 16,), jnp.float32),
            pltpu.VMEM((16,), jnp.float32),
            pltpu.VMEM((16,), jnp.float32),
            pltpu.VMEM_SHARED((16 * 16,), jnp.float32),
            pltpu.VMEM_SHARED((16 * 16,), jnp.float32),
            pltpu.SemaphoreType.DMA,
        ],
    )


def kernel(chosen_tokens, probs, time_step_mask, delta_reward):
    return _build_rl_loss_sc()(chosen_tokens.astype(jnp.int32), time_step_mask,
                               delta_reward, probs)


# speedup vs baseline: 8.8032x; 1.0380x over previous
import functools

import jax
import jax.numpy as jnp
from jax import lax
from jax.experimental import pallas as pl
from jax.experimental.pallas import tpu as pltpu
from jax.experimental.pallas import tpu_sc as plsc

B = 16
T = 128
V = 32000
TPW = T // 2
NG = TPW // 16

_LN2 = 0.6931471805599453
_SQRT2 = 1.4142135623730951


def _neg_log(x):
    bits = lax.bitcast_convert_type(x, jnp.int32)
    e = lax.shift_right_logical(bits, 23) - 127
    mbits = jnp.bitwise_or(jnp.bitwise_and(bits, 0x007FFFFF), 0x3F800000)
    m = lax.bitcast_convert_type(mbits, jnp.float32)
    big = m > _SQRT2
    m = jnp.where(big, m * 0.5, m)
    ef = e.astype(jnp.float32) + jnp.where(big, 1.0, 0.0)
    z = (m - 1.0) / (m + 1.0)
    z2 = z * z
    p = 1.0 + z2 * (1.0 / 3.0 + z2 * (1.0 / 5.0 + z2 * (1.0 / 7.0 + z2 * (1.0 / 9.0))))
    return -(2.0 * z * p + ef * _LN2)


def _rl_loss_body(chosen, mask, delta, probs, out,
                  cvb, mvb, buf, stgl, stgn, bigl, bign, dv, ov,
                  sh_loss, sh_ntok, sem, sem2, sem3):
    c = lax.axis_index("c")
    s = lax.axis_index("s")
    b = 8 * c + lax.shift_right_logical(s, 1)
    base_t = TPW * jnp.bitwise_and(s, 1)

    chosen_h = pltpu.async_copy(chosen.at[b, pl.ds(base_t, TPW)], cvb, sem2)
    mask_h = pltpu.async_copy(mask.at[b, pl.ds(base_t, TPW)], mvb, sem2)

    @pl.when(s == 0)
    def _prefetch_delta():
        pltpu.async_copy(delta, dv, sem3)

    chosen_h.wait()

    for g in range(NG):
        cv = cvb[pl.ds(16 * g, 16)]
        for j in range(16):
            cj = cv[j]
            v0 = pl.multiple_of(jnp.bitwise_and(cj, 0x7FF0), 16)
            t = 16 * g + j
            pltpu.async_copy(probs.at[b, base_t + t, pl.ds(v0, 16)],
                             buf.at[pl.ds(16 * t, 16)], sem)
    mask_h.wait()
    pltpu.make_async_copy(probs.at[0, 0, pl.ds(0, TPW * 16)], buf, sem).wait()

    lane = lax.iota(jnp.int32, 16)
    acc = jnp.zeros((16,), jnp.float32)
    nacc = jnp.zeros((16,), jnp.float32)
    for g in range(NG):
        cv = cvb[pl.ds(16 * g, 16)]
        cols = jnp.bitwise_and(cv, 15)
        m = mvb[pl.ds(16 * g, 16)]
        sel_a = jnp.zeros((16,), jnp.float32)
        sel_b = jnp.zeros((16,), jnp.float32)
        for j in range(16):
            chunk = buf[pl.ds(16 * (16 * g + j), 16)]
            gj = chunk[jnp.full((16,), cols[j], jnp.int32)]
            if j % 2 == 0:
                sel_a = jnp.where(lane == j, gj, sel_a)
            else:
                sel_b = jnp.where(lane == j, gj, sel_b)
        acc = acc + _neg_log(sel_a + sel_b) * m
        nacc = nacc + m

    def _lane_sum(x):
        for sh in (8, 4, 2, 1):
            x = x + x[jnp.bitwise_xor(lane, sh)]
        return x

    stgl[...] = jnp.where(lane == b, _lane_sum(acc), 0.0)
    stgn[...] = jnp.where(lane == b, _lane_sum(nacc), 0.0)
    pltpu.sync_copy(stgl, sh_loss.at[pl.ds(16 * s, 16)])
    pltpu.sync_copy(stgn, sh_ntok.at[pl.ds(16 * s, 16)])
    plsc.subcore_barrier()

    @pl.when(s == 0)
    def _finalize():
        pltpu.sync_copy(sh_loss, bigl)
        pltpu.sync_copy(sh_ntok, bign)
        pltpu.make_async_copy(delta, dv, sem3).wait()
        lt = jnp.zeros((16,), jnp.float32)
        nt = jnp.zeros((16,), jnp.float32)
        for k in range(16):
            lt = lt + bigl[pl.ds(16 * k, 16)]
            nt = nt + bign[pl.ds(16 * k, 16)]
        ov[...] = lt * dv[...] / nt
        half = pl.multiple_of(8 * c, 8)
        pltpu.sync_copy(ov.at[pl.ds(half, 8)], out.at[pl.ds(half, 8)])


@functools.cache
def _build_rl_loss_sc():
    return pl.kernel(
        _rl_loss_body,
        out_type=jax.ShapeDtypeStruct((B,), jnp.float32),
        mesh=plsc.VectorSubcoreMesh(core_axis_name="c", subcore_axis_name="s",
                                    num_cores=2),
        scratch_types=[
            pltpu.VMEM((TPW,), jnp.int32),
            pltpu.VMEM((TPW,), jnp.float32),
            pltpu.VMEM((TPW * 16,), jnp.float32),
            pltpu.VMEM((16,), jnp.float32),
            pltpu.VMEM((16,), jnp.float32),
            pltpu.VMEM((16 * 16,), jnp.float32),
            pltpu.VMEM((16 * 16,), jnp.float32),
            pltpu.VMEM((16,), jnp.float32),
            pltpu.VMEM((16,), jnp.float32),
            pltpu.VMEM_SHARED((16 * 16,), jnp.float32),
            pltpu.VMEM_SHARED((16 * 16,), jnp.float32),
            pltpu.SemaphoreType.DMA,
            pltpu.SemaphoreType.DMA,
            pltpu.SemaphoreType.DMA,
        ],
    )


def kernel(chosen_tokens, probs, time_step_mask, delta_reward):
    return _build_rl_loss_sc()(chosen_tokens.astype(jnp.int32), time_step_mask,
                               delta_reward, probs)


# speedup vs baseline: 8.8422x; 1.0044x over previous
import functools

import jax
import jax.numpy as jnp
from jax import lax
from jax.experimental import pallas as pl
from jax.experimental.pallas import tpu as pltpu
from jax.experimental.pallas import tpu_sc as plsc

B = 16
T = 128
V = 32000
TPW = T // 2
NG = TPW // 16

_LN2 = 0.6931471805599453
_SQRT2 = 1.4142135623730951


def _neg_log(x):
    bits = lax.bitcast_convert_type(x, jnp.int32)
    e = lax.shift_right_logical(bits, 23) - 127
    mbits = jnp.bitwise_or(jnp.bitwise_and(bits, 0x007FFFFF), 0x3F800000)
    m = lax.bitcast_convert_type(mbits, jnp.float32)
    big = m > _SQRT2
    m = jnp.where(big, m * 0.5, m)
    ef = e.astype(jnp.float32) + jnp.where(big, 1.0, 0.0)
    z = (m - 1.0) / (m + 1.0)
    z2 = z * z
    p = 1.0 + z2 * (1.0 / 3.0 + z2 * (1.0 / 5.0 + z2 * (1.0 / 7.0 + z2 * (1.0 / 9.0))))
    return -(2.0 * z * p + ef * _LN2)


def _rl_loss_body(chosen, mask, delta, probs, out,
                  cvb, mvb, buf, stg, big, dv, ov,
                  sh, sem, sem2, sem3):
    c = lax.axis_index("c")
    s = lax.axis_index("s")
    b = 8 * c + lax.shift_right_logical(s, 1)
    base_t = TPW * jnp.bitwise_and(s, 1)

    chosen_h = pltpu.async_copy(chosen.at[b, pl.ds(base_t, TPW)], cvb, sem2)
    mask_h = pltpu.async_copy(mask.at[b, pl.ds(base_t, TPW)], mvb, sem2)

    @pl.when(s == 0)
    def _prefetch_delta():
        pltpu.async_copy(delta, dv, sem3)

    chosen_h.wait()

    for g in range(NG):
        cv = cvb[pl.ds(16 * g, 16)]
        for j in range(16):
            cj = cv[j]
            v0 = pl.multiple_of(jnp.bitwise_and(cj, 0x7FF0), 16)
            t = 16 * g + j
            pltpu.async_copy(probs.at[b, base_t + t, pl.ds(v0, 16)],
                             buf.at[pl.ds(16 * t, 16)], sem)
    mask_h.wait()
    pltpu.make_async_copy(probs.at[0, 0, pl.ds(0, TPW * 16)], buf, sem).wait()

    lane = lax.iota(jnp.int32, 16)
    acc = jnp.zeros((16,), jnp.float32)
    nacc = jnp.zeros((16,), jnp.float32)
    for g in range(NG):
        cv = cvb[pl.ds(16 * g, 16)]
        cols = jnp.bitwise_and(cv, 15)
        m = mvb[pl.ds(16 * g, 16)]
        sel_a = jnp.zeros((16,), jnp.float32)
        sel_b = jnp.zeros((16,), jnp.float32)
        for j in range(16):
            chunk = buf[pl.ds(16 * (16 * g + j), 16)]
            gj = chunk[jnp.full((16,), cols[j], jnp.int32)]
            if j % 2 == 0:
                sel_a = jnp.where(lane == j, gj, sel_a)
            else:
                sel_b = jnp.where(lane == j, gj, sel_b)
        acc = acc + _neg_log(sel_a + sel_b) * m
        nacc = nacc + m

    def _lane_sum(x):
        for sh in (8, 4, 2, 1):
            x = x + x[jnp.bitwise_xor(lane, sh)]
        return x

    stg[pl.ds(0, 16)] = jnp.where(lane == b, _lane_sum(acc), 0.0)
    stg[pl.ds(16, 16)] = jnp.where(lane == b, _lane_sum(nacc), 0.0)
    pltpu.sync_copy(stg, sh.at[pl.ds(32 * s, 32)])
    plsc.subcore_barrier()

    @pl.when(s == 0)
    def _finalize():
        pltpu.sync_copy(sh, big)
        pltpu.make_async_copy(delta, dv, sem3).wait()
        lt = jnp.zeros((16,), jnp.float32)
        nt = jnp.zeros((16,), jnp.float32)
        for k in range(16):
            lt = lt + big[pl.ds(32 * k, 16)]
            nt = nt + big[pl.ds(32 * k + 16, 16)]
        ov[...] = lt * dv[...] / nt
        half = pl.multiple_of(8 * c, 8)
        pltpu.sync_copy(ov.at[pl.ds(half, 8)], out.at[pl.ds(half, 8)])


@functools.cache
def _build_rl_loss_sc():
    return pl.kernel(
        _rl_loss_body,
        out_type=jax.ShapeDtypeStruct((B,), jnp.float32),
        mesh=plsc.VectorSubcoreMesh(core_axis_name="c", subcore_axis_name="s",
                                    num_cores=2),
        scratch_types=[
            pltpu.VMEM((TPW,), jnp.int32),
            pltpu.VMEM((TPW,), jnp.float32),
            pltpu.VMEM((TPW * 16,), jnp.float32),
            pltpu.VMEM((32,), jnp.float32),
            pltpu.VMEM((32 * 16,), jnp.float32),
            pltpu.VMEM((16,), jnp.float32),
            pltpu.VMEM((16,), jnp.float32),
            pltpu.VMEM_SHARED((32 * 16,), jnp.float32),
            pltpu.SemaphoreType.DMA,
            pltpu.SemaphoreType.DMA,
            pltpu.SemaphoreType.DMA,
        ],
    )


def kernel(chosen_tokens, probs, time_step_mask, delta_reward):
    return _build_rl_loss_sc()(chosen_tokens.astype(jnp.int32), time_step_mask,
                               delta_reward, probs)
